# baseline (speedup 1.0000x reference)
import functools

import jax
import jax.numpy as jnp
import numpy as np
from jax import lax
from jax.experimental import pallas as pl
from jax.experimental.pallas import tpu as pltpu
from jax.experimental.pallas import tpu_sc as plsc

EPS = 1e-5
L = 16

B = 16
N = 2048
NFEAT = 64
NP1, NS1 = 32, 8
NP2, NS2 = 16, 8
R2 = 1.0
D1 = 128
D2 = 1152


def _lane_iota():
    return lax.iota(jnp.int32, L)


def _splat_i32(x):
    return jnp.full((L,), x, jnp.int32)


def _splat_f32(x):
    return jnp.full((L,), x, jnp.float32)


def _fps(xv, yv, zv, distv, fpsv, npts, npoint, uf):
    nchunk = npts // L
    lane = _lane_iota()

    def init_chunk(j, _):
        for u in range(uf):
            distv[pl.ds((j * uf + u) * L, L)] = _splat_f32(1e10)
        return 0

    lax.fori_loop(0, nchunk // uf, init_chunk, 0)

    def step(s, far):
        plsc.store_scatter(fpsv, [_splat_i32(s)], _splat_i32(far),
                           mask=lane == 0)
        farv = _splat_i32(far)
        cx = plsc.load_gather(xv, [farv])
        cy = plsc.load_gather(yv, [farv])
        cz = plsc.load_gather(zv, [farv])

        def chunk(jj, carry):
            bmax, bchunk = carry
            for u in range(uf):
                j = jj * uf + u
                sl = pl.ds(j * L, L)
                xs = xv[sl]
                ys = yv[sl]
                zs = zv[sl]
                dd = ((xs - cx) * (xs - cx) + (ys - cy) * (ys - cy)) \
                    + (zs - cz) * (zs - cz)
                dmin = jnp.minimum(distv[sl], dd)
                distv[sl] = dmin
                upd = dmin > bmax
                bmax = jnp.where(upd, dmin, bmax)
                bchunk = jnp.where(upd, _splat_i32(j), bchunk)
            return bmax, bchunk

        bmax, bchunk = lax.fori_loop(
            0, nchunk // uf, chunk,
            (_splat_f32(-jnp.inf), _splat_i32(0)))
        m = jnp.max(bmax)
        cand = jnp.where(bmax == m, bchunk * L + lane, _splat_i32(npts))
        return jnp.min(cand)

    lax.fori_loop(0, npoint, step, jnp.int32(0))


def _fps_split(xv, yv, zv, distv, fpsv, mshare, mloc, h, sid, npts, npoint,
               uf):
    half = npts // 2
    nchunk = half // L
    lane = _lane_iota()

    def init_chunk(j, _):
        for u in range(uf):
            distv[pl.ds((j * uf + u) * L, L)] = _splat_f32(1e10)
        return 0

    lax.fori_loop(0, nchunk // uf, init_chunk, 0)

    def step(s, far):
        plsc.store_scatter(fpsv, [_splat_i32(s)], _splat_i32(far),
                           mask=lane == 0)
        farv = _splat_i32(far)
        cx = plsc.load_gather(xv, [farv])
        cy = plsc.load_gather(yv, [farv])
        cz = plsc.load_gather(zv, [farv])

        def chunk(jj, carry):
            bmax, bchunk = carry
            for u in range(uf):
                j = jj * uf + u
                xs = xv[pl.ds(h * half + j * L, L)]
                ys = yv[pl.ds(h * half + j * L, L)]
                zs = zv[pl.ds(h * half + j * L, L)]
                dd = ((xs - cx) * (xs - cx) + (ys - cy) * (ys - cy)) \
                    + (zs - cz) * (zs - cz)
                dmin = jnp.minimum(distv[pl.ds(j * L, L)], dd)
                distv[pl.ds(j * L, L)] = dmin
                upd = dmin > bmax
                bmax = jnp.where(upd, dmin, bmax)
                bchunk = jnp.where(upd, _splat_i32(j), bchunk)
            return bmax, bchunk

        bmax, bchunk = lax.fori_loop(
            0, nchunk // uf, chunk,
            (_splat_f32(-jnp.inf), _splat_i32(0)))
        m = jnp.max(bmax)
        cand = jnp.where(bmax == m, h * half + bchunk * L + lane,
                         _splat_i32(npts))
        gi = jnp.min(cand)

        p = s % 2
        mloc[...] = jnp.where(lane == 0, jnp.full((L,), m),
                              jnp.where(lane == 1,
                                        jnp.full((L,), gi.astype(jnp.float32)),
                                        _splat_f32(0.0)))
        pltpu.sync_copy(mloc, mshare.at[pl.ds((p * L + sid) * L, L)])
        plsc.subcore_barrier()
        pltpu.sync_copy(mshare.at[pl.ds((p * L + (sid ^ 1)) * L, L)], mloc)
        other = mloc[...]
        m2 = other[0]
        g2 = other[1].astype(jnp.int32)
        far2 = jnp.where(m2 > m, g2,
                         jnp.where(m2 == m, jnp.minimum(gi, g2), gi))
        return far2

    lax.fori_loop(0, npoint, step, jnp.int32(0))


def _ball_query(xv, yv, zv, cxs, cys, czs, cbuf, glistv, npts, s_base,
                n_cent, row_base):
    nchunk = npts // L
    lane = _lane_iota()

    def one(t, _):
        sv = _splat_i32(s_base + t)
        cx = plsc.load_gather(cxs, [sv])
        cy = plsc.load_gather(cys, [sv])
        cz = plsc.load_gather(czs, [sv])

        def cond(carry):
            j, c = carry
            return jnp.logical_and(j < nchunk, c < NS1)

        def chunk(carry):
            j, c = carry
            sl = pl.ds(j * L, L)
            xs = xv[sl]
            ys = yv[sl]
            zs = zv[sl]
            dd = ((xs - cx) * (xs - cx) + (ys - cy) * (ys - cy)) \
                + (zs - cz) * (zs - cz)
            mask = jnp.logical_not(dd > R2)
            gi = j * L + lane
            cs = jnp.minimum(c, NS1)
            plsc.store_compressed(cbuf.at[pl.ds(cs, L)], gi, mask=mask)
            cnt = plsc.all_reduce_population_count(mask)[0]
            return j + 1, c + cnt

        _, c = lax.while_loop(cond, chunk, (jnp.int32(0), jnp.int32(0)))
        row = cbuf[pl.ds(0, L)]
        first = plsc.load_gather(cbuf, [_splat_i32(0)])
        fixed = jnp.where(lane < jnp.minimum(c, NS1), row, first)
        plsc.store_scatter(glistv, [t * NS1 + lane],
                           fixed + _splat_i32(row_base), mask=lane < NS1)
        return 0

    lax.fori_loop(0, n_cent, one, 0)


def _sc_stage1(xyz, table1):
    mesh = plsc.VectorSubcoreMesh(core_axis_name="c", subcore_axis_name="s",
                                  num_cores=2, num_subcores=16)
    n_rows1 = B * NP1 * NS1

    @functools.partial(
        pl.kernel,
        out_type=(
            jax.ShapeDtypeStruct((n_rows1, D1), jnp.float32),
            jax.ShapeDtypeStruct((B * 3 * NP1,), jnp.float32),
            jax.ShapeDtypeStruct((B * NP2 * NS2,), jnp.int32),
            jax.ShapeDtypeStruct((B * 3 * NP2,), jnp.float32),
        ),
        mesh=mesh,
        scratch_types=[
            pltpu.VMEM((N,), jnp.float32),
            pltpu.VMEM((N,), jnp.float32),
            pltpu.VMEM((N,), jnp.float32),
            pltpu.VMEM((N,), jnp.float32),
            pltpu.VMEM((NP1,), jnp.int32),
            pltpu.VMEM((NP1,), jnp.float32),
            pltpu.VMEM((NP1,), jnp.float32),
            pltpu.VMEM((NP1,), jnp.float32),
            pltpu.VMEM((NS1 + L,), jnp.int32),
            pltpu.VMEM((L * NS1,), jnp.int32),
            pltpu.VMEM((L * NS1, D1), jnp.float32),
            pltpu.VMEM((NP2,), jnp.int32),
            pltpu.VMEM((NP2,), jnp.float32),
            pltpu.VMEM((NP2,), jnp.float32),
            pltpu.VMEM((NP2,), jnp.float32),
            pltpu.VMEM_SHARED((2 * L * L,), jnp.float32),
            pltpu.VMEM((L,), jnp.float32),
            pltpu.SemaphoreType.DMA,
        ],
        compiler_params=pltpu.CompilerParams(needs_layout_passes=False),
    )
    def k(xyz_hbm, table_hbm, x1_hbm, cent1_hbm, glist2_hbm, cent2_hbm,
          xv, yv, zv, distv, fpsv, cxs, cys, czs, cbuf,
          glistv, rows, fps2v, cx2s, cy2s, cz2s, mshare, mloc, sem):
        sid = lax.axis_index("s")
        wid = lax.axis_index("c") * 16 + sid
        b = wid // 2
        h = wid % 2
        lane = _lane_iota()

        pltpu.sync_copy(xyz_hbm.at[pl.ds(b * 3 * N, N)], xv)
        pltpu.sync_copy(xyz_hbm.at[pl.ds(b * 3 * N + N, N)], yv)
        pltpu.sync_copy(xyz_hbm.at[pl.ds(b * 3 * N + 2 * N, N)], zv)

        _fps_split(xv, yv, zv, distv, fpsv, mshare, mloc, h, sid, N, NP1,
                   uf=4)

        for kk in range(NP1 // L):
            sl = pl.ds(kk * L, L)
            idxk = fpsv[sl]
            cxs[sl] = plsc.load_gather(xv, [idxk])
            cys[sl] = plsc.load_gather(yv, [idxk])
            czs[sl] = plsc.load_gather(zv, [idxk])

        _ball_query(xv, yv, zv, cxs, cys, czs, cbuf, glistv, N,
                    h * L, L, b * N)

        pltpu.async_copy(table_hbm.at[glistv], rows, sem).wait()

        def subcols(g, _):
            p16 = g * L + lane
            sidx = h * L + jnp.right_shift(p16, 3)
            for ci, cs in ((0, cxs), (1, cys), (2, czs)):
                civ = _splat_i32(ci)
                col = plsc.load_gather(rows, [p16, civ])
                cg = plsc.load_gather(cs, [sidx])
                plsc.store_scatter(rows, [p16, civ], col - cg)
            return 0

        lax.fori_loop(0, (L * NS1) // L, subcols, 0)

        pltpu.sync_copy(rows, x1_hbm.at[pl.ds(b * NP1 * NS1 + h * L * NS1,
                                              L * NS1)])

        @pl.when(h == 0)
        def _():
            pltpu.sync_copy(cxs, cent1_hbm.at[pl.ds(b * 3 * NP1, NP1)])
            pltpu.sync_copy(cys, cent1_hbm.at[pl.ds(b * 3 * NP1 + NP1, NP1)])
            pltpu.sync_copy(czs, cent1_hbm.at[pl.ds(b * 3 * NP1 + 2 * NP1,
                                                    NP1)])

            _fps(cxs, cys, czs, distv, fps2v, NP1, NP2, uf=2)
            idx2 = fps2v[pl.ds(0, L)]
            cx2s[pl.ds(0, L)] = plsc.load_gather(cxs, [idx2])
            cy2s[pl.ds(0, L)] = plsc.load_gather(cys, [idx2])
            cz2s[pl.ds(0, L)] = plsc.load_gather(czs, [idx2])

            _ball_query(cxs, cys, czs, cx2s, cy2s, cz2s, cbuf, glistv,
                        NP1, 0, NP2, b * NP1)
            pltpu.sync_copy(glistv,
                            glist2_hbm.at[pl.ds(b * NP2 * NS2, NP2 * NS2)])
            pltpu.sync_copy(cx2s, cent2_hbm.at[pl.ds(b * 3 * NP2, NP2)])
            pltpu.sync_copy(cy2s, cent2_hbm.at[pl.ds(b * 3 * NP2 + NP2, NP2)])
            pltpu.sync_copy(cz2s, cent2_hbm.at[pl.ds(b * 3 * NP2 + 2 * NP2,
                                                     NP2)])

    x1, cent1, glist2, cent2 = k(xyz.reshape(-1), table1)
    return (x1, cent1.reshape(B, 3, NP1), glist2.reshape(B, NP2 * NS2),
            cent2.reshape(B, 3, NP2))


def _sc_gather2(table2, glist2, cent2):
    mesh = plsc.VectorSubcoreMesh(core_axis_name="c", subcore_axis_name="s",
                                  num_cores=2, num_subcores=16)
    n_rows2 = B * NP2 * NS2
    half = (NP2 * NS2) // 2

    @functools.partial(
        pl.kernel,
        out_type=jax.ShapeDtypeStruct((n_rows2, D2), jnp.float32),
        mesh=mesh,
        scratch_types=[
            pltpu.VMEM((half,), jnp.int32),
            pltpu.VMEM((half, D2), jnp.float32),
            pltpu.VMEM((NP2,), jnp.float32),
            pltpu.VMEM((NP2,), jnp.float32),
            pltpu.VMEM((NP2,), jnp.float32),
            pltpu.SemaphoreType.DMA,
        ],
        compiler_params=pltpu.CompilerParams(needs_layout_passes=False),
    )
    def k(table_hbm, glist_hbm, cent_hbm, x2_hbm, idxv, rows, cx2, cy2, cz2,
          sem):
        wid = lax.axis_index("s") * 2 + lax.axis_index("c")
        b = wid // 2
        h = wid % 2
        lane = _lane_iota()

        pltpu.sync_copy(glist_hbm.at[pl.ds(b * NP2 * NS2 + h * half, half)],
                        idxv)
        pltpu.sync_copy(cent_hbm.at[pl.ds(b * 3 * NP2, NP2)], cx2)
        pltpu.sync_copy(cent_hbm.at[pl.ds(b * 3 * NP2 + NP2, NP2)], cy2)
        pltpu.sync_copy(cent_hbm.at[pl.ds(b * 3 * NP2 + 2 * NP2, NP2)], cz2)
        pltpu.async_copy(table_hbm.at[idxv], rows, sem).wait()

        def subcols(g, _):
            p16 = g * L + lane
            sidx = jnp.right_shift(h * half + p16, 3)
            for ci, cs in ((0, cx2), (1, cy2), (2, cz2)):
                civ = _splat_i32(ci)
                col = plsc.load_gather(rows, [p16, civ])
                cg = plsc.load_gather(cs, [sidx])
                plsc.store_scatter(rows, [p16, civ], col - cg)
            return 0

        lax.fori_loop(0, half // L, subcols, 0)
        pltpu.sync_copy(rows, x2_hbm.at[pl.ds(b * NP2 * NS2 + h * half,
                                              half)])

    return k(table2, glist2.reshape(-1), cent2.reshape(-1))


def _mlp_pool_kernel(x_ref, w1_ref, b1_ref, w2_ref, b2_ref, w3_ref, b3_ref,
                     out_ref, *, pool):
    x = x_ref[...]
    h = jnp.maximum(
        jnp.dot(x, w1_ref[...], preferred_element_type=jnp.float32)
        + b1_ref[...], 0.0)
    h = jnp.maximum(
        jnp.dot(h, w2_ref[...], preferred_element_type=jnp.float32)
        + b2_ref[...], 0.0)
    h = jnp.maximum(
        jnp.dot(h, w3_ref[...], preferred_element_type=jnp.float32)
        + b3_ref[...], 0.0)
    npts, c = h.shape
    out_ref[...] = jnp.max(h.reshape(npts // pool, pool, c), axis=1)


def _mlp_pool(x, ws, bs, pool, block):
    p, cin = x.shape
    grid = p // block
    c1, c2, c3 = ws[0].shape[1], ws[1].shape[1], ws[2].shape[1]
    return pl.pallas_call(
        functools.partial(_mlp_pool_kernel, pool=pool),
        grid=(grid,),
        in_specs=[
            pl.BlockSpec((block, cin), lambda i: (i, 0)),
            pl.BlockSpec((cin, c1), lambda i: (0, 0)),
            pl.BlockSpec((1, c1), lambda i: (0, 0)),
            pl.BlockSpec((c1, c2), lambda i: (0, 0)),
            pl.BlockSpec((1, c2), lambda i: (0, 0)),
            pl.BlockSpec((c2, c3), lambda i: (0, 0)),
            pl.BlockSpec((1, c3), lambda i: (0, 0)),
        ],
        out_specs=pl.BlockSpec((block // pool, c3), lambda i: (i, 0)),
        out_shape=jax.ShapeDtypeStruct((p // pool, c3), jnp.float32),
    )(x, ws[0], bs[0], ws[1], bs[1], ws[2], bs[2])


def _mlp3_head_kernel(x_ref, w1_ref, b1_ref, w2_ref, b2_ref, w3_ref, b3_ref,
                      f1w_ref, f1b_ref, f2w_ref, f2b_ref, f3w_ref, f3b_ref,
                      out_ref):
    x = x_ref[...]
    h = jnp.maximum(
        jnp.dot(x, w1_ref[...], preferred_element_type=jnp.float32)
        + b1_ref[...], 0.0)
    h = jnp.maximum(
        jnp.dot(h, w2_ref[...], preferred_element_type=jnp.float32)
        + b2_ref[...], 0.0)
    h = jnp.maximum(
        jnp.dot(h, w3_ref[...], preferred_element_type=jnp.float32)
        + b3_ref[...], 0.0)
    npts, c = h.shape
    g = jnp.max(h.reshape(B, npts // B, c), axis=1)
    y = jnp.maximum(
        jnp.dot(g, f1w_ref[...], preferred_element_type=jnp.float32)
        + f1b_ref[...], 0.0)
    y = jnp.maximum(
        jnp.dot(y, f2w_ref[...], preferred_element_type=jnp.float32)
        + f2b_ref[...], 0.0)
    logit = jnp.dot(y, f3w_ref[...], preferred_element_type=jnp.float32) \
        + f3b_ref[...]
    out_ref[...] = 1.0 / (1.0 + jnp.exp(-logit))


def _fold_bn(lyr):
    s = lyr['gamma'] / jnp.sqrt(1.0 + EPS)
    w = lyr['W'].T * s[None, :]
    bb = lyr['b'] * s + lyr['beta']
    return w, bb.reshape(1, -1)


def _pad_rows(w, rows):
    return jnp.pad(w, ((0, rows - w.shape[0]), (0, 0)))


def kernel(xyz, feat, params):
    xyz = xyz.astype(jnp.float32)
    feat = feat.astype(jnp.float32)

    xyz_t = jnp.transpose(xyz, (0, 2, 1))
    feat_t = jnp.transpose(feat, (0, 2, 1))
    table1 = jnp.concatenate(
        [xyz_t, feat_t, jnp.zeros((B, N, D1 - 3 - NFEAT), jnp.float32)],
        axis=-1).reshape(B * N, D1)

    x1, cent1, glist2, cent2 = _sc_stage1(xyz, table1)

    sa1 = [_fold_bn(l) for l in params['sa1']]
    sa2 = [_fold_bn(l) for l in params['sa2']]
    sa3 = [_fold_bn(l) for l in params['sa3']]
    w1a = _pad_rows(sa1[0][0], D1)
    w2a = _pad_rows(sa2[0][0], D2)
    w3a = _pad_rows(sa3[0][0], D2)

    l1 = _mlp_pool(x1, [w1a, sa1[1][0], sa1[2][0]],
                   [sa1[0][1], sa1[1][1], sa1[2][1]], pool=NS1, block=512)

    cent1_rows = jnp.transpose(cent1, (0, 2, 1)).reshape(B * NP1, 3)
    table2 = jnp.concatenate(
        [cent1_rows, l1, jnp.zeros((B * NP1, D2 - 3 - 1024), jnp.float32)],
        axis=-1)

    x2 = _sc_gather2(table2, glist2, cent2)

    l2 = _mlp_pool(x2, [w2a, sa2[1][0], sa2[2][0]],
                   [sa2[0][1], sa2[1][1], sa2[2][1]], pool=NS2, block=512)

    cent2_rows = jnp.transpose(cent2, (0, 2, 1)).reshape(B * NP2, 3)
    x3 = jnp.concatenate(
        [cent2_rows, l2, jnp.zeros((B * NP2, D2 - 3 - 1024), jnp.float32)],
        axis=-1)

    sbn = 1.0 / jnp.sqrt(1.0 + EPS)
    f1w = params['fc1W'].T * (params['bn1g'] * sbn)[None, :]
    f1b = (params['fc1b'] * params['bn1g'] * sbn
           + params['bn1b']).reshape(1, -1)
    f2w = params['fc2W'].T * (params['bn2g'] * sbn)[None, :]
    f2b = (params['fc2b'] * params['bn2g'] * sbn
           + params['bn2b']).reshape(1, -1)
    f3w = params['fc3W'].T
    f3b = params['fc3b'].reshape(1, -1)

    out = pl.pallas_call(
        _mlp3_head_kernel,
        out_shape=jax.ShapeDtypeStruct((B, 1), jnp.float32),
    )(x3, w3a, sa3[0][1], sa3[1][0], sa3[1][1], sa3[2][0], sa3[2][1],
      f1w, f1b, f2w, f2b, f3w, f3b)
    return out


# speedup vs baseline: 1.0786x; 1.0786x over previous
import functools

import jax
import jax.numpy as jnp
import numpy as np
from jax import lax
from jax.experimental import pallas as pl
from jax.experimental.pallas import tpu as pltpu
from jax.experimental.pallas import tpu_sc as plsc

EPS = 1e-5
L = 16

B = 16
N = 2048
NFEAT = 64
NP1, NS1 = 32, 8
NP2, NS2 = 16, 8
R2 = 1.0
D1 = 128
D2 = 1152


def _lane_iota():
    return lax.iota(jnp.int32, L)


def _splat_i32(x):
    return jnp.full((L,), x, jnp.int32)


def _splat_f32(x):
    return jnp.full((L,), x, jnp.float32)


def _fps(xv, yv, zv, distv, fpsv, npts, npoint, uf):
    nchunk = npts // L
    lane = _lane_iota()

    def init_chunk(j, _):
        for u in range(uf):
            distv[pl.ds((j * uf + u) * L, L)] = _splat_f32(1e10)
        return 0

    lax.fori_loop(0, nchunk // uf, init_chunk, 0)

    def step(s, far):
        plsc.store_scatter(fpsv, [_splat_i32(s)], _splat_i32(far),
                           mask=lane == 0)
        farv = _splat_i32(far)
        cx = plsc.load_gather(xv, [farv])
        cy = plsc.load_gather(yv, [farv])
        cz = plsc.load_gather(zv, [farv])

        def chunk(jj, carry):
            bmax, bchunk = carry
            for u in range(uf):
                j = jj * uf + u
                sl = pl.ds(j * L, L)
                xs = xv[sl]
                ys = yv[sl]
                zs = zv[sl]
                dd = ((xs - cx) * (xs - cx) + (ys - cy) * (ys - cy)) \
                    + (zs - cz) * (zs - cz)
                dmin = jnp.minimum(distv[sl], dd)
                distv[sl] = dmin
                upd = dmin > bmax
                bmax = jnp.where(upd, dmin, bmax)
                bchunk = jnp.where(upd, _splat_i32(j), bchunk)
            return bmax, bchunk

        bmax, bchunk = lax.fori_loop(
            0, nchunk // uf, chunk,
            (_splat_f32(-jnp.inf), _splat_i32(0)))
        m = jnp.max(bmax)
        cand = jnp.where(bmax == m, bchunk * L + lane, _splat_i32(npts))
        return jnp.min(cand)

    lax.fori_loop(0, npoint, step, jnp.int32(0))


def _fps_split(xv, yv, zv, distv, fpsv, mshare, mloc, h, sid, npts, npoint,
               uf):
    half = npts // 2
    nchunk = half // L
    lane = _lane_iota()

    def init_chunk(j, _):
        for u in range(uf):
            distv[pl.ds((j * uf + u) * L, L)] = _splat_f32(1e10)
        return 0

    lax.fori_loop(0, nchunk // uf, init_chunk, 0)

    def step(s, far):
        plsc.store_scatter(fpsv, [_splat_i32(s)], _splat_i32(far),
                           mask=lane == 0)
        farv = _splat_i32(far)
        cx = plsc.load_gather(xv, [farv])
        cy = plsc.load_gather(yv, [farv])
        cz = plsc.load_gather(zv, [farv])

        def chunk(jj, carry):
            bmax, bchunk = carry
            for u in range(uf):
                j = jj * uf + u
                xs = xv[pl.ds(h * half + j * L, L)]
                ys = yv[pl.ds(h * half + j * L, L)]
                zs = zv[pl.ds(h * half + j * L, L)]
                dd = ((xs - cx) * (xs - cx) + (ys - cy) * (ys - cy)) \
                    + (zs - cz) * (zs - cz)
                dmin = jnp.minimum(distv[pl.ds(j * L, L)], dd)
                distv[pl.ds(j * L, L)] = dmin
                upd = dmin > bmax
                bmax = jnp.where(upd, dmin, bmax)
                bchunk = jnp.where(upd, _splat_i32(j), bchunk)
            return bmax, bchunk

        bmax, bchunk = lax.fori_loop(
            0, nchunk // uf, chunk,
            (_splat_f32(-jnp.inf), _splat_i32(0)))
        m = jnp.max(bmax)
        cand = jnp.where(bmax == m, h * half + bchunk * L + lane,
                         _splat_i32(npts))
        gi = jnp.min(cand)

        p = s % 2
        mloc[...] = jnp.where(lane == 0, jnp.full((L,), m),
                              jnp.where(lane == 1,
                                        jnp.full((L,), gi.astype(jnp.float32)),
                                        _splat_f32(0.0)))
        pltpu.sync_copy(mloc, mshare.at[pl.ds((p * L + sid) * L, L)])
        plsc.subcore_barrier()
        pltpu.sync_copy(mshare.at[pl.ds((p * L + (sid ^ 1)) * L, L)], mloc)
        other = mloc[...]
        m2 = other[0]
        g2 = other[1].astype(jnp.int32)
        far2 = jnp.where(m2 > m, g2,
                         jnp.where(m2 == m, jnp.minimum(gi, g2), gi))
        return far2

    lax.fori_loop(0, npoint, step, jnp.int32(0))


def _ball_query(xv, yv, zv, cxs, cys, czs, cbuf, glistv, npts, s_base,
                n_cent, row_base, uf):
    nchunk = npts // L
    lane = _lane_iota()

    def one(t, _):
        sv = _splat_i32(s_base + t)
        cx = plsc.load_gather(cxs, [sv])
        cy = plsc.load_gather(cys, [sv])
        cz = plsc.load_gather(czs, [sv])

        def chunk(jj, c):
            for u in range(uf):
                j = jj * uf + u
                sl = pl.ds(j * L, L)
                xs = xv[sl]
                ys = yv[sl]
                zs = zv[sl]
                dd = ((xs - cx) * (xs - cx) + (ys - cy) * (ys - cy)) \
                    + (zs - cz) * (zs - cz)
                mask = jnp.logical_not(dd > R2)
                gi = j * L + lane
                cs = jnp.minimum(c, NS1)
                plsc.store_compressed(cbuf.at[pl.ds(cs, L)], gi, mask=mask)
                cnt = plsc.all_reduce_population_count(mask)[0]
                c = c + cnt
            return c

        c = lax.fori_loop(0, nchunk // uf, chunk, jnp.int32(0))
        row = cbuf[pl.ds(0, L)]
        first = plsc.load_gather(cbuf, [_splat_i32(0)])
        fixed = jnp.where(lane < jnp.minimum(c, NS1), row, first)
        plsc.store_scatter(glistv, [t * NS1 + lane],
                           fixed + _splat_i32(row_base), mask=lane < NS1)
        return 0

    lax.fori_loop(0, n_cent, one, 0)


def _sc_stage1(xyz, table1):
    mesh = plsc.VectorSubcoreMesh(core_axis_name="c", subcore_axis_name="s",
                                  num_cores=2, num_subcores=16)
    n_rows1 = B * NP1 * NS1

    @functools.partial(
        pl.kernel,
        out_type=(
            jax.ShapeDtypeStruct((n_rows1, D1), jnp.float32),
            jax.ShapeDtypeStruct((B * 3 * NP1,), jnp.float32),
            jax.ShapeDtypeStruct((B * NP2 * NS2,), jnp.int32),
            jax.ShapeDtypeStruct((B * 3 * NP2,), jnp.float32),
        ),
        mesh=mesh,
        scratch_types=[
            pltpu.VMEM((N,), jnp.float32),
            pltpu.VMEM((N,), jnp.float32),
            pltpu.VMEM((N,), jnp.float32),
            pltpu.VMEM((N,), jnp.float32),
            pltpu.VMEM((NP1,), jnp.int32),
            pltpu.VMEM((NP1,), jnp.float32),
            pltpu.VMEM((NP1,), jnp.float32),
            pltpu.VMEM((NP1,), jnp.float32),
            pltpu.VMEM((NS1 + L,), jnp.int32),
            pltpu.VMEM((L * NS1,), jnp.int32),
            pltpu.VMEM((L * NS1, D1), jnp.float32),
            pltpu.VMEM((NP2,), jnp.int32),
            pltpu.VMEM((NP2,), jnp.float32),
            pltpu.VMEM((NP2,), jnp.float32),
            pltpu.VMEM((NP2,), jnp.float32),
            pltpu.VMEM_SHARED((2 * L * L,), jnp.float32),
            pltpu.VMEM((L,), jnp.float32),
            pltpu.SemaphoreType.DMA,
        ],
        compiler_params=pltpu.CompilerParams(needs_layout_passes=False),
    )
    def k(xyz_hbm, table_hbm, x1_hbm, cent1_hbm, glist2_hbm, cent2_hbm,
          xv, yv, zv, distv, fpsv, cxs, cys, czs, cbuf,
          glistv, rows, fps2v, cx2s, cy2s, cz2s, mshare, mloc, sem):
        sid = lax.axis_index("s")
        wid = lax.axis_index("c") * 16 + sid
        b = wid // 2
        h = wid % 2
        lane = _lane_iota()

        pltpu.sync_copy(xyz_hbm.at[pl.ds(b * 3 * N, N)], xv)
        pltpu.sync_copy(xyz_hbm.at[pl.ds(b * 3 * N + N, N)], yv)
        pltpu.sync_copy(xyz_hbm.at[pl.ds(b * 3 * N + 2 * N, N)], zv)

        _fps_split(xv, yv, zv, distv, fpsv, mshare, mloc, h, sid, N, NP1,
                   uf=4)

        for kk in range(NP1 // L):
            sl = pl.ds(kk * L, L)
            idxk = fpsv[sl]
            cxs[sl] = plsc.load_gather(xv, [idxk])
            cys[sl] = plsc.load_gather(yv, [idxk])
            czs[sl] = plsc.load_gather(zv, [idxk])

        _ball_query(xv, yv, zv, cxs, cys, czs, cbuf, glistv, N,
                    h * L, L, b * N, uf=4)

        pltpu.async_copy(table_hbm.at[glistv], rows, sem).wait()

        def subcols(g, _):
            p16 = g * L + lane
            sidx = h * L + jnp.right_shift(p16, 3)
            for ci, cs in ((0, cxs), (1, cys), (2, czs)):
                civ = _splat_i32(ci)
                col = plsc.load_gather(rows, [p16, civ])
                cg = plsc.load_gather(cs, [sidx])
                plsc.store_scatter(rows, [p16, civ], col - cg)
            return 0

        lax.fori_loop(0, (L * NS1) // L, subcols, 0)

        pltpu.sync_copy(rows, x1_hbm.at[pl.ds(b * NP1 * NS1 + h * L * NS1,
                                              L * NS1)])

        @pl.when(h == 0)
        def _():
            pltpu.sync_copy(cxs, cent1_hbm.at[pl.ds(b * 3 * NP1, NP1)])
            pltpu.sync_copy(cys, cent1_hbm.at[pl.ds(b * 3 * NP1 + NP1, NP1)])
            pltpu.sync_copy(czs, cent1_hbm.at[pl.ds(b * 3 * NP1 + 2 * NP1,
                                                    NP1)])

            _fps(cxs, cys, czs, distv, fps2v, NP1, NP2, uf=2)
            idx2 = fps2v[pl.ds(0, L)]
            cx2s[pl.ds(0, L)] = plsc.load_gather(cxs, [idx2])
            cy2s[pl.ds(0, L)] = plsc.load_gather(cys, [idx2])
            cz2s[pl.ds(0, L)] = plsc.load_gather(czs, [idx2])

            _ball_query(cxs, cys, czs, cx2s, cy2s, cz2s, cbuf, glistv,
                        NP1, 0, NP2, b * NP1, uf=2)
            pltpu.sync_copy(glistv,
                            glist2_hbm.at[pl.ds(b * NP2 * NS2, NP2 * NS2)])
            pltpu.sync_copy(cx2s, cent2_hbm.at[pl.ds(b * 3 * NP2, NP2)])
            pltpu.sync_copy(cy2s, cent2_hbm.at[pl.ds(b * 3 * NP2 + NP2, NP2)])
            pltpu.sync_copy(cz2s, cent2_hbm.at[pl.ds(b * 3 * NP2 + 2 * NP2,
                                                     NP2)])

    x1, cent1, glist2, cent2 = k(xyz.reshape(-1), table1)
    return (x1, cent1.reshape(B, 3, NP1), glist2.reshape(B, NP2 * NS2),
            cent2.reshape(B, 3, NP2))


def _sc_gather2(table2, glist2, cent2):
    mesh = plsc.VectorSubcoreMesh(core_axis_name="c", subcore_axis_name="s",
                                  num_cores=2, num_subcores=16)
    n_rows2 = B * NP2 * NS2
    half = (NP2 * NS2) // 2

    @functools.partial(
        pl.kernel,
        out_type=jax.ShapeDtypeStruct((n_rows2, D2), jnp.float32),
        mesh=mesh,
        scratch_types=[
            pltpu.VMEM((half,), jnp.int32),
            pltpu.VMEM((half, D2), jnp.float32),
            pltpu.VMEM((NP2,), jnp.float32),
            pltpu.VMEM((NP2,), jnp.float32),
            pltpu.VMEM((NP2,), jnp.float32),
            pltpu.SemaphoreType.DMA,
        ],
        compiler_params=pltpu.CompilerParams(needs_layout_passes=False),
    )
    def k(table_hbm, glist_hbm, cent_hbm, x2_hbm, idxv, rows, cx2, cy2, cz2,
          sem):
        wid = lax.axis_index("s") * 2 + lax.axis_index("c")
        b = wid // 2
        h = wid % 2
        lane = _lane_iota()

        pltpu.sync_copy(glist_hbm.at[pl.ds(b * NP2 * NS2 + h * half, half)],
                        idxv)
        pltpu.sync_copy(cent_hbm.at[pl.ds(b * 3 * NP2, NP2)], cx2)
        pltpu.sync_copy(cent_hbm.at[pl.ds(b * 3 * NP2 + NP2, NP2)], cy2)
        pltpu.sync_copy(cent_hbm.at[pl.ds(b * 3 * NP2 + 2 * NP2, NP2)], cz2)
        pltpu.async_copy(table_hbm.at[idxv], rows, sem).wait()

        def subcols(g, _):
            p16 = g * L + lane
            sidx = jnp.right_shift(h * half + p16, 3)
            for ci, cs in ((0, cx2), (1, cy2), (2, cz2)):
                civ = _splat_i32(ci)
                col = plsc.load_gather(rows, [p16, civ])
                cg = plsc.load_gather(cs, [sidx])
                plsc.store_scatter(rows, [p16, civ], col - cg)
            return 0

        lax.fori_loop(0, half // L, subcols, 0)
        pltpu.sync_copy(rows, x2_hbm.at[pl.ds(b * NP2 * NS2 + h * half,
                                              half)])

    return k(table2, glist2.reshape(-1), cent2.reshape(-1))


def _mlp_pool_kernel(x_ref, w1_ref, b1_ref, w2_ref, b2_ref, w3_ref, b3_ref,
                     out_ref, *, pool):
    x = x_ref[...]
    h = jnp.maximum(
        jnp.dot(x, w1_ref[...], preferred_element_type=jnp.float32)
        + b1_ref[...], 0.0)
    h = jnp.maximum(
        jnp.dot(h, w2_ref[...], preferred_element_type=jnp.float32)
        + b2_ref[...], 0.0)
    h = jnp.maximum(
        jnp.dot(h, w3_ref[...], preferred_element_type=jnp.float32)
        + b3_ref[...], 0.0)
    npts, c = h.shape
    out_ref[...] = jnp.max(h.reshape(npts // pool, pool, c), axis=1)


def _mlp_pool(x, ws, bs, pool, block):
    p, cin = x.shape
    grid = p // block
    c1, c2, c3 = ws[0].shape[1], ws[1].shape[1], ws[2].shape[1]
    return pl.pallas_call(
        functools.partial(_mlp_pool_kernel, pool=pool),
        grid=(grid,),
        in_specs=[
            pl.BlockSpec((block, cin), lambda i: (i, 0)),
            pl.BlockSpec((cin, c1), lambda i: (0, 0)),
            pl.BlockSpec((1, c1), lambda i: (0, 0)),
            pl.BlockSpec((c1, c2), lambda i: (0, 0)),
            pl.BlockSpec((1, c2), lambda i: (0, 0)),
            pl.BlockSpec((c2, c3), lambda i: (0, 0)),
            pl.BlockSpec((1, c3), lambda i: (0, 0)),
        ],
        out_specs=pl.BlockSpec((block // pool, c3), lambda i: (i, 0)),
        out_shape=jax.ShapeDtypeStruct((p // pool, c3), jnp.float32),
    )(x, ws[0], bs[0], ws[1], bs[1], ws[2], bs[2])


def _mlp3_head_kernel(x_ref, w1_ref, b1_ref, w2_ref, b2_ref, w3_ref, b3_ref,
                      f1w_ref, f1b_ref, f2w_ref, f2b_ref, f3w_ref, f3b_ref,
                      out_ref):
    x = x_ref[...]
    h = jnp.maximum(
        jnp.dot(x, w1_ref[...], preferred_element_type=jnp.float32)
        + b1_ref[...], 0.0)
    h = jnp.maximum(
        jnp.dot(h, w2_ref[...], preferred_element_type=jnp.float32)
        + b2_ref[...], 0.0)
    h = jnp.maximum(
        jnp.dot(h, w3_ref[...], preferred_element_type=jnp.float32)
        + b3_ref[...], 0.0)
    npts, c = h.shape
    g = jnp.max(h.reshape(B, npts // B, c), axis=1)
    y = jnp.maximum(
        jnp.dot(g, f1w_ref[...], preferred_element_type=jnp.float32)
        + f1b_ref[...], 0.0)
    y = jnp.maximum(
        jnp.dot(y, f2w_ref[...], preferred_element_type=jnp.float32)
        + f2b_ref[...], 0.0)
    logit = jnp.dot(y, f3w_ref[...], preferred_element_type=jnp.float32) \
        + f3b_ref[...]
    out_ref[...] = 1.0 / (1.0 + jnp.exp(-logit))


def _fold_bn(lyr):
    s = lyr['gamma'] / jnp.sqrt(1.0 + EPS)
    w = lyr['W'].T * s[None, :]
    bb = lyr['b'] * s + lyr['beta']
    return w, bb.reshape(1, -1)


def _pad_rows(w, rows):
    return jnp.pad(w, ((0, rows - w.shape[0]), (0, 0)))


def kernel(xyz, feat, params):
    xyz = xyz.astype(jnp.float32)
    feat = feat.astype(jnp.float32)

    xyz_t = jnp.transpose(xyz, (0, 2, 1))
    feat_t = jnp.transpose(feat, (0, 2, 1))
    table1 = jnp.concatenate(
        [xyz_t, feat_t, jnp.zeros((B, N, D1 - 3 - NFEAT), jnp.float32)],
        axis=-1).reshape(B * N, D1)

    x1, cent1, glist2, cent2 = _sc_stage1(xyz, table1)

    sa1 = [_fold_bn(l) for l in params['sa1']]
    sa2 = [_fold_bn(l) for l in params['sa2']]
    sa3 = [_fold_bn(l) for l in params['sa3']]
    w1a = _pad_rows(sa1[0][0], D1)
    w2a = _pad_rows(sa2[0][0], D2)
    w3a = _pad_rows(sa3[0][0], D2)

    l1 = _mlp_pool(x1, [w1a, sa1[1][0], sa1[2][0]],
                   [sa1[0][1], sa1[1][1], sa1[2][1]], pool=NS1, block=512)

    cent1_rows = jnp.transpose(cent1, (0, 2, 1)).reshape(B * NP1, 3)
    table2 = jnp.concatenate(
        [cent1_rows, l1, jnp.zeros((B * NP1, D2 - 3 - 1024), jnp.float32)],
        axis=-1)

    x2 = _sc_gather2(table2, glist2, cent2)

    l2 = _mlp_pool(x2, [w2a, sa2[1][0], sa2[2][0]],
                   [sa2[0][1], sa2[1][1], sa2[2][1]], pool=NS2, block=512)

    cent2_rows = jnp.transpose(cent2, (0, 2, 1)).reshape(B * NP2, 3)
    x3 = jnp.concatenate(
        [cent2_rows, l2, jnp.zeros((B * NP2, D2 - 3 - 1024), jnp.float32)],
        axis=-1)

    sbn = 1.0 / jnp.sqrt(1.0 + EPS)
    f1w = params['fc1W'].T * (params['bn1g'] * sbn)[None, :]
    f1b = (params['fc1b'] * params['bn1g'] * sbn
           + params['bn1b']).reshape(1, -1)
    f2w = params['fc2W'].T * (params['bn2g'] * sbn)[None, :]
    f2b = (params['fc2b'] * params['bn2g'] * sbn
           + params['bn2b']).reshape(1, -1)
    f3w = params['fc3W'].T
    f3b = params['fc3b'].reshape(1, -1)

    out = pl.pallas_call(
        _mlp3_head_kernel,
        out_shape=jax.ShapeDtypeStruct((B, 1), jnp.float32),
    )(x3, w3a, sa3[0][1], sa3[1][0], sa3[1][1], sa3[2][0], sa3[2][1],
      f1w, f1b, f2w, f2b, f3w, f3b)
    return out


# speedup vs baseline: 1.0896x; 1.0102x over previous
import functools

import jax
import jax.numpy as jnp
import numpy as np
from jax import lax
from jax.experimental import pallas as pl
from jax.experimental.pallas import tpu as pltpu
from jax.experimental.pallas import tpu_sc as plsc

EPS = 1e-5
L = 16

B = 16
N = 2048
NFEAT = 64
NP1, NS1 = 32, 8
NP2, NS2 = 16, 8
R2 = 1.0
D1 = 128
D2 = 1152


def _lane_iota():
    return lax.iota(jnp.int32, L)


def _splat_i32(x):
    return jnp.full((L,), x, jnp.int32)


def _splat_f32(x):
    return jnp.full((L,), x, jnp.float32)


def _fps(xv, yv, zv, distv, fpsv, npts, npoint, uf):
    nchunk = npts // L
    lane = _lane_iota()

    def init_chunk(j, _):
        for u in range(uf):
            distv[pl.ds((j * uf + u) * L, L)] = _splat_f32(1e10)
        return 0

    lax.fori_loop(0, nchunk // uf, init_chunk, 0)

    def step(s, far):
        plsc.store_scatter(fpsv, [_splat_i32(s)], _splat_i32(far),
                           mask=lane == 0)
        farv = _splat_i32(far)
        cx = plsc.load_gather(xv, [farv])
        cy = plsc.load_gather(yv, [farv])
        cz = plsc.load_gather(zv, [farv])

        def chunk(jj, carry):
            bmax, bchunk = carry
            for u in range(uf):
                j = jj * uf + u
                sl = pl.ds(j * L, L)
                xs = xv[sl]
                ys = yv[sl]
                zs = zv[sl]
                dd = ((xs - cx) * (xs - cx) + (ys - cy) * (ys - cy)) \
                    + (zs - cz) * (zs - cz)
                dmin = jnp.minimum(distv[sl], dd)
                distv[sl] = dmin
                upd = dmin > bmax
                bmax = jnp.where(upd, dmin, bmax)
                bchunk = jnp.where(upd, _splat_i32(j), bchunk)
            return bmax, bchunk

        bmax, bchunk = lax.fori_loop(
            0, nchunk // uf, chunk,
            (_splat_f32(-jnp.inf), _splat_i32(0)))
        m = jnp.max(bmax)
        cand = jnp.where(bmax == m, bchunk * L + lane, _splat_i32(npts))
        return jnp.min(cand)

    lax.fori_loop(0, npoint, step, jnp.int32(0))


def _fps_split(xv, yv, zv, distv, fpsv, mshare, mloc, h, sid, npts, npoint,
               uf):
    half = npts // 2
    nchunk = half // L
    lane = _lane_iota()

    def init_chunk(j, _):
        for u in range(uf):
            distv[pl.ds((j * uf + u) * L, L)] = _splat_f32(1e10)
        return 0

    lax.fori_loop(0, nchunk // uf, init_chunk, 0)

    def step(s, far):
        plsc.store_scatter(fpsv, [_splat_i32(s)], _splat_i32(far),
                           mask=lane == 0)
        farv = _splat_i32(far)
        cx = plsc.load_gather(xv, [farv])
        cy = plsc.load_gather(yv, [farv])
        cz = plsc.load_gather(zv, [farv])

        def chunk(jj, carry):
            bmax, bchunk = carry
            for u in range(uf):
                j = jj * uf + u
                xs = xv[pl.ds(h * half + j * L, L)]
                ys = yv[pl.ds(h * half + j * L, L)]
                zs = zv[pl.ds(h * half + j * L, L)]
                dd = ((xs - cx) * (xs - cx) + (ys - cy) * (ys - cy)) \
                    + (zs - cz) * (zs - cz)
                dmin = jnp.minimum(distv[pl.ds(j * L, L)], dd)
                distv[pl.ds(j * L, L)] = dmin
                upd = dmin > bmax
                bmax = jnp.where(upd, dmin, bmax)
                bchunk = jnp.where(upd, _splat_i32(j), bchunk)
            return bmax, bchunk

        bmax, bchunk = lax.fori_loop(
            0, nchunk // uf, chunk,
            (_splat_f32(-jnp.inf), _splat_i32(0)))
        m = jnp.max(bmax)
        cand = jnp.where(bmax == m, h * half + bchunk * L + lane,
                         _splat_i32(npts))
        gi = jnp.min(cand)

        p = s % 2
        mloc[...] = jnp.where(lane == 0, jnp.full((L,), m),
                              jnp.where(lane == 1,
                                        jnp.full((L,), gi.astype(jnp.float32)),
                                        _splat_f32(0.0)))
        pltpu.sync_copy(mloc, mshare.at[pl.ds((p * L + sid) * L, L)])
        plsc.subcore_barrier()
        pltpu.sync_copy(mshare.at[pl.ds((p * L + (sid ^ 1)) * L, L)], mloc)
        other = mloc[...]
        m2 = other[0]
        g2 = other[1].astype(jnp.int32)
        far2 = jnp.where(m2 > m, g2,
                         jnp.where(m2 == m, jnp.minimum(gi, g2), gi))
        return far2

    lax.fori_loop(0, npoint, step, jnp.int32(0))


def _ball_query(xv, yv, zv, cxs, cys, czs, cbuf, glistv, npts, s_base,
                n_cent, row_base):
    nchunk = npts // L
    lane = _lane_iota()

    def one(t, _):
        sv = _splat_i32(s_base + t)
        cx = plsc.load_gather(cxs, [sv])
        cy = plsc.load_gather(cys, [sv])
        cz = plsc.load_gather(czs, [sv])

        def chunk(j, c):
            sl = pl.ds(j * L, L)
            xs = xv[sl]
            ys = yv[sl]
            zs = zv[sl]
            dd = ((xs - cx) * (xs - cx) + (ys - cy) * (ys - cy)) \
                + (zs - cz) * (zs - cz)
            mask = jnp.logical_not(dd > R2)
            gi = j * L + lane
            cs = jnp.minimum(c, NS1)
            plsc.store_compressed(cbuf.at[pl.ds(cs, L)], gi, mask=mask)
            cnt = plsc.all_reduce_population_count(mask)[0]
            return c + cnt

        c = lax.fori_loop(0, nchunk, chunk, jnp.int32(0))
        row = cbuf[pl.ds(0, L)]
        first = plsc.load_gather(cbuf, [_splat_i32(0)])
        fixed = jnp.where(lane < jnp.minimum(c, NS1), row, first)
        plsc.store_scatter(glistv, [t * NS1 + lane],
                           fixed + _splat_i32(row_base), mask=lane < NS1)
        return 0

    lax.fori_loop(0, n_cent, one, 0)


def _sc_stage1(xyz, table1):
    mesh = plsc.VectorSubcoreMesh(core_axis_name="c", subcore_axis_name="s",
                                  num_cores=2, num_subcores=16)
    n_rows1 = B * NP1 * NS1

    @functools.partial(
        pl.kernel,
        out_type=(
            jax.ShapeDtypeStruct((n_rows1, D1), jnp.float32),
            jax.ShapeDtypeStruct((B * 3 * NP1,), jnp.float32),
            jax.ShapeDtypeStruct((B * NP2 * NS2,), jnp.int32),
            jax.ShapeDtypeStruct((B * 3 * NP2,), jnp.float32),
        ),
        mesh=mesh,
        scratch_types=[
            pltpu.VMEM((N,), jnp.float32),
            pltpu.VMEM((N,), jnp.float32),
            pltpu.VMEM((N,), jnp.float32),
            pltpu.VMEM((N,), jnp.float32),
            pltpu.VMEM((NP1,), jnp.int32),
            pltpu.VMEM((NP1,), jnp.float32),
            pltpu.VMEM((NP1,), jnp.float32),
            pltpu.VMEM((NP1,), jnp.float32),
            pltpu.VMEM((NS1 + L,), jnp.int32),
            pltpu.VMEM((L * NS1,), jnp.int32),
            pltpu.VMEM((L * NS1, D1), jnp.float32),
            pltpu.VMEM((NP2,), jnp.int32),
            pltpu.VMEM((NP2,), jnp.float32),
            pltpu.VMEM((NP2,), jnp.float32),
            pltpu.VMEM((NP2,), jnp.float32),
            pltpu.VMEM_SHARED((2 * L * L,), jnp.float32),
            pltpu.VMEM((L,), jnp.float32),
            pltpu.SemaphoreType.DMA,
        ],
        compiler_params=pltpu.CompilerParams(needs_layout_passes=False),
    )
    def k(xyz_hbm, table_hbm, x1_hbm, cent1_hbm, glist2_hbm, cent2_hbm,
          xv, yv, zv, distv, fpsv, cxs, cys, czs, cbuf,
          glistv, rows, fps2v, cx2s, cy2s, cz2s, mshare, mloc, sem):
        sid = lax.axis_index("s")
        wid = lax.axis_index("c") * 16 + sid
        b = wid // 2
        h = wid % 2
        lane = _lane_iota()

        pltpu.sync_copy(xyz_hbm.at[pl.ds(b * 3 * N, N)], xv)
        pltpu.sync_copy(xyz_hbm.at[pl.ds(b * 3 * N + N, N)], yv)
        pltpu.sync_copy(xyz_hbm.at[pl.ds(b * 3 * N + 2 * N, N)], zv)

        _fps_split(xv, yv, zv, distv, fpsv, mshare, mloc, h, sid, N, NP1,
                   uf=4)

        for kk in range(NP1 // L):
            sl = pl.ds(kk * L, L)
            idxk = fpsv[sl]
            cxs[sl] = plsc.load_gather(xv, [idxk])
            cys[sl] = plsc.load_gather(yv, [idxk])
            czs[sl] = plsc.load_gather(zv, [idxk])

        _ball_query(xv, yv, zv, cxs, cys, czs, cbuf, glistv, N,
                    h * L, L, b * N)

        pltpu.async_copy(table_hbm.at[glistv], rows, sem).wait()

        def subcols(g, _):
            p16 = g * L + lane
            sidx = h * L + jnp.right_shift(p16, 3)
            for ci, cs in ((0, cxs), (1, cys), (2, czs)):
                civ = _splat_i32(ci)
                col = plsc.load_gather(rows, [p16, civ])
                cg = plsc.load_gather(cs, [sidx])
                plsc.store_scatter(rows, [p16, civ], col - cg)
            return 0

        lax.fori_loop(0, (L * NS1) // L, subcols, 0)

        pltpu.sync_copy(rows, x1_hbm.at[pl.ds(b * NP1 * NS1 + h * L * NS1,
                                              L * NS1)])

        @pl.when(h == 0)
        def _():
            pltpu.sync_copy(cxs, cent1_hbm.at[pl.ds(b * 3 * NP1, NP1)])
            pltpu.sync_copy(cys, cent1_hbm.at[pl.ds(b * 3 * NP1 + NP1, NP1)])
            pltpu.sync_copy(czs, cent1_hbm.at[pl.ds(b * 3 * NP1 + 2 * NP1,
                                                    NP1)])

            _fps(cxs, cys, czs, distv, fps2v, NP1, NP2, uf=2)
            idx2 = fps2v[pl.ds(0, L)]
            cx2s[pl.ds(0, L)] = plsc.load_gather(cxs, [idx2])
            cy2s[pl.ds(0, L)] = plsc.load_gather(cys, [idx2])
            cz2s[pl.ds(0, L)] = plsc.load_gather(czs, [idx2])

            _ball_query(cxs, cys, czs, cx2s, cy2s, cz2s, cbuf, glistv,
                        NP1, 0, NP2, b * NP1)
            pltpu.sync_copy(glistv,
                            glist2_hbm.at[pl.ds(b * NP2 * NS2, NP2 * NS2)])
            pltpu.sync_copy(cx2s, cent2_hbm.at[pl.ds(b * 3 * NP2, NP2)])
            pltpu.sync_copy(cy2s, cent2_hbm.at[pl.ds(b * 3 * NP2 + NP2, NP2)])
            pltpu.sync_copy(cz2s, cent2_hbm.at[pl.ds(b * 3 * NP2 + 2 * NP2,
                                                     NP2)])

    x1, cent1, glist2, cent2 = k(xyz.reshape(-1), table1)
    return (x1, cent1.reshape(B, 3, NP1), glist2.reshape(B, NP2 * NS2),
            cent2.reshape(B, 3, NP2))


def _sc_gather2(table2, glist2, cent2):
    mesh = plsc.VectorSubcoreMesh(core_axis_name="c", subcore_axis_name="s",
                                  num_cores=2, num_subcores=16)
    n_rows2 = B * NP2 * NS2
    half = (NP2 * NS2) // 2

    @functools.partial(
        pl.kernel,
        out_type=jax.ShapeDtypeStruct((n_rows2, D2), jnp.float32),
        mesh=mesh,
        scratch_types=[
            pltpu.VMEM((half,), jnp.int32),
            pltpu.VMEM((half, D2), jnp.float32),
            pltpu.VMEM((NP2,), jnp.float32),
            pltpu.VMEM((NP2,), jnp.float32),
            pltpu.VMEM((NP2,), jnp.float32),
            pltpu.SemaphoreType.DMA,
        ],
        compiler_params=pltpu.CompilerParams(needs_layout_passes=False),
    )
    def k(table_hbm, glist_hbm, cent_hbm, x2_hbm, idxv, rows, cx2, cy2, cz2,
          sem):
        wid = lax.axis_index("s") * 2 + lax.axis_index("c")
        b = wid // 2
        h = wid % 2
        lane = _lane_iota()

        pltpu.sync_copy(glist_hbm.at[pl.ds(b * NP2 * NS2 + h * half, half)],
                        idxv)
        pltpu.sync_copy(cent_hbm.at[pl.ds(b * 3 * NP2, NP2)], cx2)
        pltpu.sync_copy(cent_hbm.at[pl.ds(b * 3 * NP2 + NP2, NP2)], cy2)
        pltpu.sync_copy(cent_hbm.at[pl.ds(b * 3 * NP2 + 2 * NP2, NP2)], cz2)
        pltpu.async_copy(table_hbm.at[idxv], rows, sem).wait()

        def subcols(g, _):
            p16 = g * L + lane
            sidx = jnp.right_shift(h * half + p16, 3)
            for ci, cs in ((0, cx2), (1, cy2), (2, cz2)):
                civ = _splat_i32(ci)
                col = plsc.load_gather(rows, [p16, civ])
                cg = plsc.load_gather(cs, [sidx])
                plsc.store_scatter(rows, [p16, civ], col - cg)
            return 0

        lax.fori_loop(0, half // L, subcols, 0)
        pltpu.sync_copy(rows, x2_hbm.at[pl.ds(b * NP2 * NS2 + h * half,
                                              half)])

    return k(table2, glist2.reshape(-1), cent2.reshape(-1))


def _mlp_pool_kernel(x_ref, w1_ref, b1_ref, w2_ref, b2_ref, w3_ref, b3_ref,
                     out_ref, *, pool):
    x = x_ref[...]
    h = jnp.maximum(
        jnp.dot(x, w1_ref[...], preferred_element_type=jnp.float32)
        + b1_ref[...], 0.0)
    h = jnp.maximum(
        jnp.dot(h, w2_ref[...], preferred_element_type=jnp.float32)
        + b2_ref[...], 0.0)
    h = jnp.maximum(
        jnp.dot(h, w3_ref[...], preferred_element_type=jnp.float32)
        + b3_ref[...], 0.0)
    npts, c = h.shape
    out_ref[...] = jnp.max(h.reshape(npts // pool, pool, c), axis=1)


def _mlp_pool(x, ws, bs, pool, block):
    p, cin = x.shape
    grid = p // block
    c1, c2, c3 = ws[0].shape[1], ws[1].shape[1], ws[2].shape[1]
    return pl.pallas_call(
        functools.partial(_mlp_pool_kernel, pool=pool),
        grid=(grid,),
        in_specs=[
            pl.BlockSpec((block, cin), lambda i: (i, 0)),
            pl.BlockSpec((cin, c1), lambda i: (0, 0)),
            pl.BlockSpec((1, c1), lambda i: (0, 0)),
            pl.BlockSpec((c1, c2), lambda i: (0, 0)),
            pl.BlockSpec((1, c2), lambda i: (0, 0)),
            pl.BlockSpec((c2, c3), lambda i: (0, 0)),
            pl.BlockSpec((1, c3), lambda i: (0, 0)),
        ],
        out_specs=pl.BlockSpec((block // pool, c3), lambda i: (i, 0)),
        out_shape=jax.ShapeDtypeStruct((p // pool, c3), jnp.float32),
    )(x, ws[0], bs[0], ws[1], bs[1], ws[2], bs[2])


def _mlp3_head_kernel(x_ref, w1_ref, b1_ref, w2_ref, b2_ref, w3_ref, b3_ref,
                      f1w_ref, f1b_ref, f2w_ref, f2b_ref, f3w_ref, f3b_ref,
                      out_ref):
    x = x_ref[...]
    h = jnp.maximum(
        jnp.dot(x, w1_ref[...], preferred_element_type=jnp.float32)
        + b1_ref[...], 0.0)
    h = jnp.maximum(
        jnp.dot(h, w2_ref[...], preferred_element_type=jnp.float32)
        + b2_ref[...], 0.0)
    h = jnp.maximum(
        jnp.dot(h, w3_ref[...], preferred_element_type=jnp.float32)
        + b3_ref[...], 0.0)
    npts, c = h.shape
    g = jnp.max(h.reshape(B, npts // B, c), axis=1)
    y = jnp.maximum(
        jnp.dot(g, f1w_ref[...], preferred_element_type=jnp.float32)
        + f1b_ref[...], 0.0)
    y = jnp.maximum(
        jnp.dot(y, f2w_ref[...], preferred_element_type=jnp.float32)
        + f2b_ref[...], 0.0)
    logit = jnp.dot(y, f3w_ref[...], preferred_element_type=jnp.float32) \
        + f3b_ref[...]
    out_ref[...] = 1.0 / (1.0 + jnp.exp(-logit))


def _fold_bn(lyr):
    s = lyr['gamma'] / jnp.sqrt(1.0 + EPS)
    w = lyr['W'].T * s[None, :]
    bb = lyr['b'] * s + lyr['beta']
    return w, bb.reshape(1, -1)


def _pad_rows(w, rows):
    return jnp.pad(w, ((0, rows - w.shape[0]), (0, 0)))


def kernel(xyz, feat, params):
    xyz = xyz.astype(jnp.float32)
    feat = feat.astype(jnp.float32)

    xyz_t = jnp.transpose(xyz, (0, 2, 1))
    feat_t = jnp.transpose(feat, (0, 2, 1))
    table1 = jnp.concatenate(
        [xyz_t, feat_t, jnp.zeros((B, N, D1 - 3 - NFEAT), jnp.float32)],
        axis=-1).reshape(B * N, D1)

    x1, cent1, glist2, cent2 = _sc_stage1(xyz, table1)

    sa1 = [_fold_bn(l) for l in params['sa1']]
    sa2 = [_fold_bn(l) for l in params['sa2']]
    sa3 = [_fold_bn(l) for l in params['sa3']]
    w1a = _pad_rows(sa1[0][0], D1)
    w2a = _pad_rows(sa2[0][0], D2)
    w3a = _pad_rows(sa3[0][0], D2)

    l1 = _mlp_pool(x1, [w1a, sa1[1][0], sa1[2][0]],
                   [sa1[0][1], sa1[1][1], sa1[2][1]], pool=NS1, block=512)

    cent1_rows = jnp.transpose(cent1, (0, 2, 1)).reshape(B * NP1, 3)
    table2 = jnp.concatenate(
        [cent1_rows, l1, jnp.zeros((B * NP1, D2 - 3 - 1024), jnp.float32)],
        axis=-1)

    x2 = _sc_gather2(table2, glist2, cent2)

    l2 = _mlp_pool(x2, [w2a, sa2[1][0], sa2[2][0]],
                   [sa2[0][1], sa2[1][1], sa2[2][1]], pool=NS2, block=512)

    cent2_rows = jnp.transpose(cent2, (0, 2, 1)).reshape(B * NP2, 3)
    x3 = jnp.concatenate(
        [cent2_rows, l2, jnp.zeros((B * NP2, D2 - 3 - 1024), jnp.float32)],
        axis=-1)

    sbn = 1.0 / jnp.sqrt(1.0 + EPS)
    f1w = params['fc1W'].T * (params['bn1g'] * sbn)[None, :]
    f1b = (params['fc1b'] * params['bn1g'] * sbn
           + params['bn1b']).reshape(1, -1)
    f2w = params['fc2W'].T * (params['bn2g'] * sbn)[None, :]
    f2b = (params['fc2b'] * params['bn2g'] * sbn
           + params['bn2b']).reshape(1, -1)
    f3w = params['fc3W'].T
    f3b = params['fc3b'].reshape(1, -1)

    out = pl.pallas_call(
        _mlp3_head_kernel,
        out_shape=jax.ShapeDtypeStruct((B, 1), jnp.float32),
    )(x3, w3a, sa3[0][1], sa3[1][0], sa3[1][1], sa3[2][0], sa3[2][1],
      f1w, f1b, f2w, f2b, f3w, f3b)
    return out
